# baseline (speedup 1.0000x reference)
import functools

import jax
import jax.numpy as jnp
from jax import lax
from jax.experimental import pallas as pl
from jax.experimental.pallas import tpu as pltpu
from jax.experimental.pallas import tpu_sc as plsc

_D = 64
_NC = 2
_NS = 16
_NW = _NC * _NS
_CH = 128
_NBUF = 4
_NWB = 2


def _build_gather(n_units, seq, d):
    per_w = n_units // _NW
    nbc = 16384 // _CH
    mesh = plsc.VectorSubcoreMesh(core_axis_name="c", subcore_axis_name="s")

    @functools.partial(
        pl.kernel,
        mesh=mesh,
        out_type=jax.ShapeDtypeStruct((seq, d // 8, nbc, 8, _CH), jnp.float32),
        scratch_types=(
            [pltpu.VMEM((per_w, _CH), jnp.int32)]
            + [pltpu.VMEM((_CH, d), jnp.float32)] * _NBUF
            + [pltpu.VMEM((d, _CH), jnp.float32)] * _NWB
            + [pltpu.SemaphoreType.DMA] * (_NBUF + _NWB)
        ),
        compiler_params=pltpu.CompilerParams(needs_layout_passes=False, use_tc_tiling_on_sc=False),
    )
    def gather_kernel(idx_hbm, table_hbm, out_hbm, idx_v, *bufs_and_sems):
        gbuf = bufs_and_sems[:_NBUF]
        wbuf = bufs_and_sems[_NBUF:_NBUF + _NWB]
        gsem = bufs_and_sems[_NBUF + _NWB:2 * _NBUF + _NWB]
        wsem = bufs_and_sems[2 * _NBUF + _NWB:]
        w = lax.axis_index("s") * _NC + lax.axis_index("c")
        u0 = w * per_w
        pltpu.sync_copy(idx_hbm.at[w], idx_v)
        for b in range(_NBUF):
            pltpu.async_copy(table_hbm.at[idx_v.at[b]], gbuf[b], gsem[b])

        iota16 = lax.iota(jnp.int32, 16)
        rows_v = [bg * 16 + iota16 for bg in range(8)]
        zeros16 = jnp.full((16,), 0, jnp.int32)

        def transpose_unit(g, wb):
            def fblk(k, carry):
                f0 = k * 8
                for df in range(8):
                    f = f0 + df
                    cols = zeros16 + f
                    for bg in range(8):
                        v = plsc.load_gather(g, [rows_v[bg], cols])
                        wb[f, pl.ds(bg * 16, 16)] = v
                return carry
            lax.fori_loop(0, d // 8, fblk, 0)

        def group(grp, carry):
            for b in range(_NBUF):
                j = grp * _NBUF + b
                u = u0 + j
                s = u // nbc
                bc = u % nbc
                ws = b % _NWB
                pltpu.make_async_copy(
                    table_hbm.at[pl.ds(0, _CH)], gbuf[b], gsem[b]).wait()
                @pl.when(j >= _NWB)
                def _():
                    for fr in range(8):
                        pltpu.make_async_copy(
                            wbuf[ws].at[pl.ds(fr * 8, 8)],
                            out_hbm.at[0, fr, 0], wsem[ws]).wait()
                transpose_unit(gbuf[b], wbuf[ws])
                nxt = j + _NBUF

                @pl.when(nxt < per_w)
                def _():
                    pltpu.async_copy(
                        table_hbm.at[idx_v.at[nxt]], gbuf[b], gsem[b])
                for fr in range(8):
                    pltpu.async_copy(
                        wbuf[ws].at[pl.ds(fr * 8, 8)],
                        out_hbm.at[s, fr, bc], wsem[ws])
            return carry

        lax.fori_loop(0, per_w // _NBUF, group, 0)
        for ws in range(_NWB):
            for fr in range(8):
                pltpu.make_async_copy(
                    wbuf[ws].at[pl.ds(fr * 8, 8)],
                    out_hbm.at[0, fr, 0], wsem[ws]).wait()

    return gather_kernel


def kernel(poi_idx, poi_embedding):
    bsz, seq = poi_idx.shape
    d = poi_embedding.shape[1]
    idx_t = jnp.transpose(poi_idx, (1, 0)).astype(jnp.int32)
    n_units = (bsz * seq) // _CH
    idx3 = jnp.reshape(idx_t, (_NW, n_units // _NW, _CH))
    out5 = _build_gather(n_units, seq, d)(idx3, poi_embedding)
    out = jnp.transpose(out5, (2, 4, 0, 1, 3))
    return jnp.reshape(out, (bsz, seq, d))
